# baseline (speedup 1.0000x reference)
import functools

import jax
import jax.numpy as jnp
from jax import lax
from jax.experimental import pallas as pl
from jax.experimental.pallas import tpu as pltpu
from jax.experimental.pallas import tpu_sc as plsc

NW = 32
CB = 8
LP = 208
S0 = 104
S1 = 96


def _pooling_sc(x, table, B, L, E):
    RPS = B // NW
    NCH = RPS // CB

    mesh = plsc.VectorSubcoreMesh(core_axis_name="c", subcore_axis_name="s")

    @functools.partial(
        pl.kernel,
        mesh=mesh,
        out_type=jax.ShapeDtypeStruct((B, E), jnp.float32),
        scratch_types=[
            pltpu.VMEM((2, CB, LP), jnp.int32),
            pltpu.VMEM((2, CB, L, E), jnp.float32),
            pltpu.VMEM((RPS, E), jnp.float32),
            pltpu.SemaphoreType.DMA,
            pltpu.SemaphoreType.DMA,
            pltpu.SemaphoreType.DMA,
        ],
        compiler_params=pltpu.CompilerParams(use_tc_tiling_on_sc=False),
    )
    def pool_kernel(x_hbm, t_hbm, out_hbm, idx_v, rows_v, acc_v, sem0, sem1,
                    isem):
        wid = lax.axis_index("s") * 2 + lax.axis_index("c")
        rbase = wid * RPS
        inv_l = jnp.float32(1.0 / L)
        sems = (sem0, sem1)

        def fire(c, buf):
            for b in range(CB):
                pltpu.async_copy(
                    t_hbm.at[idx_v.at[buf, b, pl.ds(0, S0)]],
                    rows_v.at[buf, b, pl.ds(0, S0)], sems[buf])
                pltpu.async_copy(
                    t_hbm.at[idx_v.at[buf, b, pl.ds(S0, S1)]],
                    rows_v.at[buf, b, pl.ds(S0, S1)], sems[buf])

        def drain(buf):
            for b in range(CB):
                pltpu.make_async_copy(
                    t_hbm.at[pl.ds(0, S0)],
                    rows_v.at[buf, b, pl.ds(0, S0)], sems[buf]).wait()
                pltpu.make_async_copy(
                    t_hbm.at[pl.ds(0, S1)],
                    rows_v.at[buf, b, pl.ds(S0, S1)], sems[buf]).wait()

        def prefetch_idx(c, buf):
            pltpu.async_copy(
                x_hbm.at[pl.ds(rbase + c * CB, CB)],
                idx_v.at[buf, :, pl.ds(0, L)], isem)

        def wait_idx():
            pltpu.make_async_copy(
                x_hbm.at[pl.ds(0, CB)], idx_v.at[0, :, pl.ds(0, L)],
                isem).wait()

        def reduce_chunk(c, buf):
            for b in range(CB):
                def red(i, carry, b=b, buf=buf):
                    a0, a1, a2, a3 = carry
                    l = i * 40
                    for u in range(10):
                        a0 = a0 + rows_v[buf, b, l + u, :]
                    for u in range(10):
                        a1 = a1 + rows_v[buf, b, l + 10 + u, :]
                    for u in range(10):
                        a2 = a2 + rows_v[buf, b, l + 20 + u, :]
                    for u in range(10):
                        a3 = a3 + rows_v[buf, b, l + 30 + u, :]
                    return a0, a1, a2, a3

                z = jnp.zeros((E,), jnp.float32)
                a0, a1, a2, a3 = lax.fori_loop(0, L // 40, red, (z, z, z, z))
                acc_v[c * CB + b, :] = ((a0 + a1) + (a2 + a3)) * inv_l

        pltpu.sync_copy(x_hbm.at[pl.ds(rbase, CB)], idx_v.at[0, :, pl.ds(0, L)])
        fire(0, 0)
        prefetch_idx(1, 1)

        def pair(p, carry):
            c0 = 2 * p
            c1 = 2 * p + 1

            wait_idx()
            fire(c1, 1)
            drain(0)

            @pl.when(c1 + 1 < NCH)
            def _():
                prefetch_idx(c1 + 1, 0)

            reduce_chunk(c0, 0)

            @pl.when(c1 + 1 < NCH)
            def _():
                wait_idx()
                fire(c1 + 1, 0)

            drain(1)

            @pl.when(c1 + 2 < NCH)
            def _():
                prefetch_idx(c1 + 2, 1)

            reduce_chunk(c1, 1)
            return carry

        lax.fori_loop(0, NCH // 2, pair, 0)
        pltpu.sync_copy(acc_v, out_hbm.at[pl.ds(wid * RPS, RPS)])

    return pool_kernel(x, table)


def _mlp_tc(pooled, W1, b1, W2, b2, B):
    def body(p_ref, w1_ref, b1_ref, w2_ref, b2_ref, o_ref):
        h = jnp.dot(p_ref[...], w1_ref[...], preferred_element_type=jnp.float32)
        h = jnp.maximum(h + b1_ref[...], 0.0)
        z = jnp.dot(h, w2_ref[...], preferred_element_type=jnp.float32)
        z = z + b2_ref[...]
        o_ref[...] = 1.0 / (1.0 + jnp.exp(-z))

    return pl.pallas_call(
        body,
        out_shape=jax.ShapeDtypeStruct((B, 1), jnp.float32),
    )(pooled, W1, b1.reshape(1, -1), W2, b2.reshape(1, 1))


def kernel(x, table, W1, b1, W2, b2):
    B, L = x.shape
    V, E = table.shape
    t128 = jnp.reshape(table, (V * E // 128, 128))
    t128 = jax.lax.optimization_barrier(t128)
    t_lin = jnp.reshape(t128, (V, E))
    pooled = _pooling_sc(x.astype(jnp.int32), t_lin, B, L, E)
    return _mlp_tc(pooled, W1, b1, W2, b2, B)


# speedup vs baseline: 1.4049x; 1.4049x over previous
import functools

import jax
import jax.numpy as jnp
from jax import lax
from jax.experimental import pallas as pl
from jax.experimental.pallas import tpu as pltpu
from jax.experimental.pallas import tpu_sc as plsc

NW = 32
CB = 8
LP = 208
S0 = 104
S1 = 96


def _pooling_sc(x, table, B, L, E):
    RPS = B // NW
    NCH = RPS // CB

    mesh = plsc.VectorSubcoreMesh(core_axis_name="c", subcore_axis_name="s")

    @functools.partial(
        pl.kernel,
        mesh=mesh,
        out_type=jax.ShapeDtypeStruct((B, E), jnp.float32),
        scratch_types=[
            pltpu.VMEM((2, CB, LP), jnp.int32),
            pltpu.VMEM((2, CB, L, E), jnp.float32),
            pltpu.VMEM((RPS, E), jnp.float32),
            pltpu.SemaphoreType.DMA,
            pltpu.SemaphoreType.DMA,
            pltpu.SemaphoreType.DMA,
        ],
        compiler_params=pltpu.CompilerParams(use_tc_tiling_on_sc=False),
    )
    def pool_kernel(x_hbm, t_hbm, out_hbm, idx_v, rows_v, acc_v, sem0, sem1,
                    isem):
        wid = lax.axis_index("s") * 2 + lax.axis_index("c")
        rbase = wid * RPS
        inv_l = jnp.float32(1.0 / L)
        sems = (sem0, sem1)

        def fire(c, buf):
            for b in range(CB):
                pltpu.async_copy(
                    t_hbm.at[idx_v.at[buf, b, pl.ds(0, S0)]],
                    rows_v.at[buf, b, pl.ds(0, S0)], sems[buf])
                pltpu.async_copy(
                    t_hbm.at[idx_v.at[buf, b, pl.ds(S0, S1)]],
                    rows_v.at[buf, b, pl.ds(S0, S1)], sems[buf])

        def drain(buf):
            for b in range(CB):
                pltpu.make_async_copy(
                    t_hbm.at[pl.ds(0, S0)],
                    rows_v.at[buf, b, pl.ds(0, S0)], sems[buf]).wait()
                pltpu.make_async_copy(
                    t_hbm.at[pl.ds(0, S1)],
                    rows_v.at[buf, b, pl.ds(S0, S1)], sems[buf]).wait()

        def prefetch_idx(c, buf):
            pltpu.async_copy(
                x_hbm.at[pl.ds(rbase + c * CB, CB)],
                idx_v.at[buf, :, pl.ds(0, L)], isem)

        def wait_idx():
            pltpu.make_async_copy(
                x_hbm.at[pl.ds(0, CB)], idx_v.at[0, :, pl.ds(0, L)],
                isem).wait()

        def reduce_chunk(c, buf):
            for b in range(CB):
                def red(i, carry, b=b, buf=buf):
                    a0, a1, a2, a3 = carry
                    l = i * 40
                    for u in range(10):
                        a0 = a0 + rows_v[buf, b, l + u, :]
                    for u in range(10):
                        a1 = a1 + rows_v[buf, b, l + 10 + u, :]
                    for u in range(10):
                        a2 = a2 + rows_v[buf, b, l + 20 + u, :]
                    for u in range(10):
                        a3 = a3 + rows_v[buf, b, l + 30 + u, :]
                    return a0, a1, a2, a3

                z = jnp.zeros((E,), jnp.float32)
                a0, a1, a2, a3 = lax.fori_loop(0, L // 40, red, (z, z, z, z))
                acc_v[c * CB + b, :] = ((a0 + a1) + (a2 + a3)) * inv_l

        pltpu.sync_copy(x_hbm.at[pl.ds(rbase, CB)], idx_v.at[0, :, pl.ds(0, L)])
        fire(0, 0)
        prefetch_idx(1, 1)

        def pair(p, carry):
            c0 = 2 * p
            c1 = 2 * p + 1

            wait_idx()
            fire(c1, 1)
            drain(0)

            @pl.when(c1 + 1 < NCH)
            def _():
                prefetch_idx(c1 + 1, 0)

            reduce_chunk(c0, 0)

            @pl.when(c1 + 1 < NCH)
            def _():
                wait_idx()
                fire(c1 + 1, 0)

            drain(1)

            @pl.when(c1 + 2 < NCH)
            def _():
                prefetch_idx(c1 + 2, 1)

            reduce_chunk(c1, 1)
            return carry

        lax.fori_loop(0, NCH // 2, pair, 0)
        pltpu.sync_copy(acc_v, out_hbm.at[pl.ds(wid * RPS, RPS)])

    return pool_kernel(x, table)


def _mlp_tc(pooled, W1, b1, W2, b2, B):
    def body(p_ref, w1_ref, b1_ref, w2_ref, b2_ref, o_ref):
        h = jnp.dot(p_ref[...], w1_ref[...], preferred_element_type=jnp.float32)
        h = jnp.maximum(h + b1_ref[...], 0.0)
        z = jnp.dot(h, w2_ref[...], preferred_element_type=jnp.float32)
        z = z + b2_ref[...]
        o_ref[...] = 1.0 / (1.0 + jnp.exp(-z))

    return pl.pallas_call(
        body,
        out_shape=jax.ShapeDtypeStruct((B, 1), jnp.float32),
    )(pooled, W1, b1.reshape(1, -1), W2, b2.reshape(1, 1))


def kernel(x, table, W1, b1, W2, b2):
    B, L = x.shape
    V, E = table.shape
    tT = jnp.transpose(table)
    t128 = jnp.reshape(
        jnp.transpose(jnp.reshape(tT, (E, V * E // 128, 128 // E)), (1, 2, 0)),
        (V * E // 128, 128))
    t128 = jax.lax.optimization_barrier(t128)
    t_lin = jnp.reshape(t128, (V, E))
    pooled = _pooling_sc(x.astype(jnp.int32), t_lin, B, L, E)
    return _mlp_tc(pooled, W1, b1, W2, b2, B)
